# baseline (speedup 1.0000x reference)
import jax
import jax.numpy as jnp
from jax.experimental import pallas as pl
from jax.experimental.pallas import tpu as pltpu

_N = 10000
_D = 128
_BM = 200


def _gcn_body(x_ref, adj_ref, w_ref, out_ref, support_ref):
    m = pl.program_id(0)

    @pl.when(m == 0)
    def _compute_support():
        support_ref[...] = jnp.dot(
            x_ref[...], w_ref[...], preferred_element_type=jnp.float32
        ).astype(jnp.bfloat16)

    x1 = jnp.dot(adj_ref[...].astype(jnp.bfloat16), support_ref[...],
                 preferred_element_type=jnp.float32)
    x_blk = x_ref[pl.ds(m * _BM, _BM), :]
    out_ref[...] = 0.95 * x_blk + 0.05 * x1


def kernel(x, adj, W1):
    grid = (_N // _BM,)
    return pl.pallas_call(
        _gcn_body,
        grid=grid,
        in_specs=[
            pl.BlockSpec((_N, _D), lambda m: (0, 0)),
            pl.BlockSpec((_BM, _N), lambda m: (m, 0)),
            pl.BlockSpec((_D, _D), lambda m: (0, 0)),
        ],
        out_specs=pl.BlockSpec((_BM, _D), lambda m: (m, 0)),
        out_shape=jax.ShapeDtypeStruct((_N, _D), jnp.float32),
        scratch_shapes=[pltpu.VMEM((_N, _D), jnp.bfloat16)],
    )(x, adj, W1)


# speedup vs baseline: 1.0076x; 1.0076x over previous
import jax
import jax.numpy as jnp
from jax.experimental import pallas as pl
from jax.experimental.pallas import tpu as pltpu

_N = 10000
_D = 128
_BM = 80
_NBLK = _N // _BM
_K = 8


def _gcn_body(x_ref, w_ref, adj_hbm, out_ref, bufs, support_ref, sems):
    m = pl.program_id(0)

    def _copy(blk, slot):
        return pltpu.make_async_copy(
            adj_hbm.at[pl.ds(blk * _BM, _BM), :],
            bufs.at[slot],
            sems.at[slot],
        )

    @pl.when(m == 0)
    def _prologue():
        for j in range(_K - 1):
            _copy(j, j).start()
        support_ref[...] = jnp.dot(
            x_ref[...], w_ref[...], preferred_element_type=jnp.float32
        ).astype(jnp.bfloat16)

    slot = jax.lax.rem(m, _K)
    _copy(m, slot).wait()

    @pl.when(m + _K - 1 < _NBLK)
    def _prefetch():
        _copy(m + _K - 1, jax.lax.rem(m + _K - 1, _K)).start()

    x1 = jnp.dot(bufs[slot].astype(jnp.bfloat16), support_ref[...],
                 preferred_element_type=jnp.float32)
    x_blk = x_ref[pl.ds(m * _BM, _BM), :]
    out_ref[...] = 0.95 * x_blk + 0.05 * x1


def kernel(x, adj, W1):
    return pl.pallas_call(
        _gcn_body,
        grid=(_NBLK,),
        in_specs=[
            pl.BlockSpec((_N, _D), lambda m: (0, 0)),
            pl.BlockSpec((_D, _D), lambda m: (0, 0)),
            pl.BlockSpec(memory_space=pltpu.MemorySpace.HBM),
        ],
        out_specs=pl.BlockSpec((_BM, _D), lambda m: (m, 0)),
        out_shape=jax.ShapeDtypeStruct((_N, _D), jnp.float32),
        scratch_shapes=[
            pltpu.VMEM((_K, _BM, _N), jnp.float32),
            pltpu.VMEM((_N, _D), jnp.bfloat16),
            pltpu.SemaphoreType.DMA((_K,)),
        ],
    )(x, W1, adj)


# speedup vs baseline: 1.0126x; 1.0050x over previous
import jax
import jax.numpy as jnp
from jax.experimental import pallas as pl
from jax.experimental.pallas import tpu as pltpu

_N = 10000
_D = 128
_BM = 400


def _gcn_body(x_ref, adj_ref, w_ref, out_ref, support_ref):
    m = pl.program_id(0)

    @pl.when(m == 0)
    def _compute_support():
        support_ref[...] = jnp.dot(
            x_ref[...], w_ref[...], preferred_element_type=jnp.float32
        ).astype(jnp.bfloat16)

    x1 = jnp.dot(adj_ref[...].astype(jnp.bfloat16), support_ref[...],
                 preferred_element_type=jnp.float32)
    x_blk = x_ref[pl.ds(m * _BM, _BM), :]
    out_ref[...] = 0.95 * x_blk + 0.05 * x1


def kernel(x, adj, W1):
    grid = (_N // _BM,)
    return pl.pallas_call(
        _gcn_body,
        grid=grid,
        in_specs=[
            pl.BlockSpec((_N, _D), lambda m: (0, 0)),
            pl.BlockSpec((_BM, _N), lambda m: (m, 0)),
            pl.BlockSpec((_D, _D), lambda m: (0, 0)),
        ],
        out_specs=pl.BlockSpec((_BM, _D), lambda m: (m, 0)),
        out_shape=jax.ShapeDtypeStruct((_N, _D), jnp.float32),
        scratch_shapes=[pltpu.VMEM((_N, _D), jnp.bfloat16)],
    )(x, adj, W1)
